# baseline (speedup 1.0000x reference)
import jax
import jax.numpy as jnp
from jax.experimental import pallas as pl

_EPS = 1e-12


_CHUNK = 256


def _ln_kernel(x_ref, pos_ref, gamma_ref, beta_ref, o_ref):
    bs = x_ref.shape[1]
    for c in range(0, bs, _CHUNK):
        sl = pl.ds(c, _CHUNK)
        x = x_ref[:, sl, :] + pos_ref[:, sl, :]
        mean = jnp.mean(x, axis=-1, keepdims=True)
        xc = x - mean
        var = jnp.mean(xc * xc, axis=-1, keepdims=True)
        normed = xc * jax.lax.rsqrt(var + _EPS)
        o_ref[:, sl, :] = normed * gamma_ref[...] + beta_ref[...]


def kernel(input_tensor, pos_emb, gamma, beta):
    B, S, W = input_tensor.shape
    pos = pos_emb[:S]
    gamma2 = gamma.reshape(1, 1, W)
    beta2 = beta.reshape(1, 1, W)

    bs = 1024
    num_s = S // bs

    return pl.pallas_call(
        _ln_kernel,
        grid=(num_s,),
        in_specs=[
            pl.BlockSpec((B, bs, W), lambda s: (0, s, 0)),
            pl.BlockSpec((1, bs, W), lambda s: (0, s, 0)),
            pl.BlockSpec((1, 1, W), lambda s: (0, 0, 0)),
            pl.BlockSpec((1, 1, W), lambda s: (0, 0, 0)),
        ],
        out_specs=pl.BlockSpec((B, bs, W), lambda s: (0, s, 0)),
        out_shape=jax.ShapeDtypeStruct((B, S, W), input_tensor.dtype),
    )(input_tensor, pos.reshape(1, S, W), gamma2, beta2)


# speedup vs baseline: 1.0122x; 1.0122x over previous
import jax
import jax.numpy as jnp
from jax.experimental import pallas as pl
from jax.experimental.pallas import tpu as pltpu

_EPS = 1e-12


_CHUNK = 256


def _ln_kernel(x_ref, pos_ref, gamma_ref, beta_ref, o_ref):
    bs = x_ref.shape[1]
    for c in range(0, bs, _CHUNK):
        sl = pl.ds(c, _CHUNK)
        x = x_ref[:, sl, :] + pos_ref[:, sl, :]
        mean = jnp.mean(x, axis=-1, keepdims=True)
        xc = x - mean
        var = jnp.mean(xc * xc, axis=-1, keepdims=True)
        normed = xc * jax.lax.rsqrt(var + _EPS)
        o_ref[:, sl, :] = normed * gamma_ref[...] + beta_ref[...]


def kernel(input_tensor, pos_emb, gamma, beta):
    B, S, W = input_tensor.shape
    pos = pos_emb[:S]
    gamma2 = gamma.reshape(1, 1, W)
    beta2 = beta.reshape(1, 1, W)

    bs = 512
    num_s = S // bs

    return pl.pallas_call(
        _ln_kernel,
        grid=(num_s,),
        in_specs=[
            pl.BlockSpec((B, bs, W), lambda s: (0, s, 0)),
            pl.BlockSpec((1, bs, W), lambda s: (0, s, 0)),
            pl.BlockSpec((1, 1, W), lambda s: (0, 0, 0)),
            pl.BlockSpec((1, 1, W), lambda s: (0, 0, 0)),
        ],
        out_specs=pl.BlockSpec((B, bs, W), lambda s: (0, s, 0)),
        out_shape=jax.ShapeDtypeStruct((B, S, W), input_tensor.dtype),
        compiler_params=pltpu.CompilerParams(
            dimension_semantics=("parallel",),
        ),
    )(input_tensor, pos.reshape(1, S, W), gamma2, beta2)
